# Initial kernel scaffold; baseline (speedup 1.0000x reference)
import jax
import jax.numpy as jnp
from jax.experimental import pallas as pl
from jax.experimental.pallas import tpu as pltpu

N = 4096
D = 512
KNN = 80
BR = 256

_INT_MIN = jnp.int32(-(2 ** 31))


def _f32_sort_key(v):
    b = jax.lax.bitcast_convert_type(v, jnp.int32)
    return jnp.where(b < 0, jnp.bitwise_xor(jnp.bitwise_not(b), _INT_MIN), b)


def _gso_kernel(x_ref, xt_ref, out_ref, xct_ref, s_ref):
    i = pl.program_id(0)

    @pl.when(i == 0)
    def _prep():
        xt = xt_ref[...]
        mean = jnp.sum(xt, axis=0, keepdims=True) / D
        xct = xt - mean
        xct_ref[...] = xct
        d = jnp.sum(xct * xct, axis=0, keepdims=True)
        s_ref[...] = jnp.sqrt(d / (D - 1))

    xb = x_ref[...]
    mean_r = jnp.sum(xb, axis=1, keepdims=True) / D
    xcb = xb - mean_r
    d_r = jnp.sum(xcb * xcb, axis=1, keepdims=True)
    s_row = jnp.sqrt(d_r / (D - 1))

    m = jax.lax.dot_general(
        xcb, xct_ref[...],
        (((1,), (0,)), ((), ())),
        precision=jax.lax.Precision.HIGHEST,
        preferred_element_type=jnp.float32,
    )
    c = (m / (D - 1)) / s_row / s_ref[...]
    c = jnp.clip(c, -1.0, 1.0)

    keys = _f32_sort_key(c)

    def body(t, p):
        inc = jnp.left_shift(jnp.int32(1), jnp.int32(31) - t)
        cand = p + inc
        cnt = jnp.sum((keys < cand).astype(jnp.int32), axis=1, keepdims=True)
        return jnp.where(cnt <= KNN, cand, p)

    p = jax.lax.fori_loop(0, 32, body, jnp.full((BR, 1), _INT_MIN))

    colids = jax.lax.broadcasted_iota(jnp.int32, (BR, N), 1)
    minkey = jnp.min(keys, axis=1, keepdims=True)
    first_min = jnp.min(
        jnp.where(keys == minkey, colids, jnp.int32(N)), axis=1, keepdims=True)
    keep = (keys <= p) & (colids != first_min)
    out_ref[...] = jnp.where(keep, c, 0.0)


@jax.jit
def kernel(x):
    xt = x.T
    grid = (N // BR,)
    return pl.pallas_call(
        _gso_kernel,
        grid=grid,
        in_specs=[
            pl.BlockSpec((BR, D), lambda i: (i, 0)),
            pl.BlockSpec((D, N), lambda i: (0, 0)),
        ],
        out_specs=pl.BlockSpec((BR, N), lambda i: (i, 0)),
        out_shape=jax.ShapeDtypeStruct((N, N), jnp.float32),
        scratch_shapes=[
            pltpu.VMEM((D, N), jnp.float32),
            pltpu.VMEM((1, N), jnp.float32),
        ],
        compiler_params=pltpu.CompilerParams(
            dimension_semantics=("arbitrary",),
        ),
    )(x, xt)


# speedup vs baseline: 16.5150x; 16.5150x over previous
import jax
import jax.numpy as jnp
import numpy as np
from jax.experimental import pallas as pl
from jax.experimental.pallas import tpu as pltpu

N = 4096
D = 512
KNN = 80
BR = 256

_INT_MIN = np.int32(-(2 ** 31))


def _f32_sort_key(v):
    b = jax.lax.bitcast_convert_type(v, jnp.int32)
    return jnp.where(b < 0, jnp.bitwise_xor(jnp.bitwise_not(b), _INT_MIN), b)


def _gso_kernel(x_ref, xt_ref, out_ref, xct_ref, s_ref):
    i = pl.program_id(0)

    @pl.when(i == 0)
    def _prep():
        xt = xt_ref[...]
        mean = jnp.sum(xt, axis=0, keepdims=True) / D
        xctb = (xt - mean).astype(jnp.bfloat16)
        xct_ref[...] = xctb
        xctf = xctb.astype(jnp.float32)
        d = jnp.sum(xctf * xctf, axis=0, keepdims=True)
        s_ref[...] = jnp.sqrt(d / (D - 1))

    xb = x_ref[...]
    mean_r = jnp.sum(xb, axis=1, keepdims=True) / D
    xcb = (xb - mean_r).astype(jnp.bfloat16)
    xcbf = xcb.astype(jnp.float32)
    d_r = jnp.sum(xcbf * xcbf, axis=1, keepdims=True)
    s_row = jnp.sqrt(d_r / (D - 1))

    m = jax.lax.dot_general(
        xcb, xct_ref[...],
        (((1,), (0,)), ((), ())),
        preferred_element_type=jnp.float32,
    )
    c = (m / (D - 1)) / s_row / s_ref[...]
    c = jnp.clip(c, -1.0, 1.0)

    keys = _f32_sort_key(c)

    def body(t, p):
        inc = jnp.left_shift(np.int32(1), np.int32(31) - t)
        cand = p + inc
        cnt = jnp.sum((keys < cand).astype(jnp.int32), axis=1, keepdims=True)
        return jnp.where(cnt <= KNN, cand, p)

    p = jax.lax.fori_loop(0, 32, body, jnp.full((BR, 1), _INT_MIN, jnp.int32))

    colids = jax.lax.broadcasted_iota(jnp.int32, (BR, N), 1)
    minkey = jnp.min(keys, axis=1, keepdims=True)
    first_min = jnp.min(
        jnp.where(keys == minkey, colids, np.int32(N)), axis=1, keepdims=True)
    keep = (keys <= p) & (colids != first_min)
    out_ref[...] = jnp.where(keep, c, 0.0)


@jax.jit
def kernel(x):
    xt = x.T
    grid = (N // BR,)
    return pl.pallas_call(
        _gso_kernel,
        grid=grid,
        in_specs=[
            pl.BlockSpec((BR, D), lambda i: (i, 0)),
            pl.BlockSpec((D, N), lambda i: (0, 0)),
        ],
        out_specs=pl.BlockSpec((BR, N), lambda i: (i, 0)),
        out_shape=jax.ShapeDtypeStruct((N, N), jnp.float32),
        scratch_shapes=[
            pltpu.VMEM((D, N), jnp.bfloat16),
            pltpu.VMEM((1, N), jnp.float32),
        ],
        compiler_params=pltpu.CompilerParams(
            dimension_semantics=("arbitrary",),
        ),
    )(x, xt)


# speedup vs baseline: 18.3871x; 1.1134x over previous
import jax
import jax.numpy as jnp
import numpy as np
from jax.experimental import pallas as pl
from jax.experimental.pallas import tpu as pltpu

N = 4096
D = 512
KNN = 80
BR = 512

_INT_MIN = np.int32(-(2 ** 31))


def _f32_sort_key(v):
    b = jax.lax.bitcast_convert_type(v, jnp.int32)
    return jnp.where(b < 0, jnp.bitwise_xor(jnp.bitwise_not(b), _INT_MIN), b)


def _gso_kernel(x_ref, xt_ref, out_ref, xct_ref, s_ref):
    i = pl.program_id(0)

    @pl.when(i == 0)
    def _prep():
        xt = xt_ref[...]
        mean = jnp.sum(xt, axis=0, keepdims=True) / D
        xctb = (xt - mean).astype(jnp.bfloat16)
        xct_ref[...] = xctb
        xctf = xctb.astype(jnp.float32)
        d = jnp.sum(xctf * xctf, axis=0, keepdims=True)
        s_ref[...] = jnp.sqrt(d / (D - 1))

    xb = x_ref[...]
    mean_r = jnp.sum(xb, axis=1, keepdims=True) / D
    xcb = (xb - mean_r).astype(jnp.bfloat16)
    xcbf = xcb.astype(jnp.float32)
    d_r = jnp.sum(xcbf * xcbf, axis=1, keepdims=True)
    s_row = jnp.sqrt(d_r / (D - 1))

    m = jax.lax.dot_general(
        xcb, xct_ref[...],
        (((1,), (0,)), ((), ())),
        preferred_element_type=jnp.float32,
    )
    c = (m / (D - 1)) / s_row / s_ref[...]

    keys = _f32_sort_key(c)

    def body(t, p):
        inc = jnp.left_shift(np.int32(1), np.int32(31) - t)
        cand = p + inc
        cnt = jnp.sum((keys < cand).astype(jnp.int32), axis=1, keepdims=True)
        return jnp.where(cnt <= KNN, cand, p)

    p = jax.lax.fori_loop(0, 32, body, jnp.full((BR, 1), _INT_MIN, jnp.int32))

    minkey = jnp.min(keys, axis=1, keepdims=True)
    keep = (keys <= p) & (keys > minkey)
    out_ref[...] = jnp.where(keep, c, 0.0)


@jax.jit
def kernel(x):
    xt = x.T
    grid = (N // BR,)
    return pl.pallas_call(
        _gso_kernel,
        grid=grid,
        in_specs=[
            pl.BlockSpec((BR, D), lambda i: (i, 0)),
            pl.BlockSpec((D, N), lambda i: (0, 0)),
        ],
        out_specs=pl.BlockSpec((BR, N), lambda i: (i, 0)),
        out_shape=jax.ShapeDtypeStruct((N, N), jnp.float32),
        scratch_shapes=[
            pltpu.VMEM((D, N), jnp.bfloat16),
            pltpu.VMEM((1, N), jnp.float32),
        ],
        compiler_params=pltpu.CompilerParams(
            dimension_semantics=("arbitrary",),
        ),
    )(x, xt)
